# baseline (speedup 1.0000x reference)
import jax
import jax.numpy as jnp
from jax import lax
from jax.experimental import pallas as pl
from jax.experimental.pallas import tpu as pltpu
from jax.experimental.pallas import tpu_sc as plsc

B = 16384
D = 32
NC = 2
NS = 16
NW = NC * NS
BPW = B // NW
CHUNK = 128
NCHUNK = BPW // CHUNK
TROWS = 250000


def _shuffle(x, idx):
    return lax.gather(
        x, idx[:, None],
        dimension_numbers=lax.GatherDimensionNumbers(
            offset_dims=(), collapsed_slice_dims=(0,), start_index_map=(0,)),
        slice_sizes=(1,),
        mode=lax.GatherScatterMode.PROMISE_IN_BOUNDS)


def _body(uq_hbm, iq_hbm, uo_hbm, io_hbm, utab_hbm, itab_hbm, out_hbm,
          uq_v, iq_v, uo_v, io_v, urows, irows, out_v, usem, isem):
    wid = lax.axis_index("s") * NC + lax.axis_index("c")
    base = wid * BPW

    pltpu.sync_copy(uq_hbm.at[pl.ds(wid * NCHUNK, NCHUNK)], uq_v)
    pltpu.sync_copy(iq_hbm.at[pl.ds(wid * NCHUNK, NCHUNK)], iq_v)
    pltpu.sync_copy(uo_hbm.at[pl.ds(base, BPW)], uo_v)
    pltpu.sync_copy(io_hbm.at[pl.ds(base, BPW)], io_v)

    lanes = lax.iota(jnp.int32, 16)

    for c in range(NCHUNK):
        cu = pltpu.async_copy(utab_hbm.at[uq_v.at[c]], urows, usem)
        ci = pltpu.async_copy(itab_hbm.at[iq_v.at[c]], irows, isem)
        cu.wait()
        ci.wait()

        def stage(g, carry, c=c):
            acc = jnp.zeros((16,), jnp.float32)
            uoffs = uo_v[pl.ds(c * CHUNK + g * 16, 16)]
            ioffs = io_v[pl.ds(c * CHUNK + g * 16, 16)]
            for j in range(16):
                r = g * 16 + j
                uoff = pl.multiple_of(uoffs[j], 32)
                ioff = pl.multiple_of(ioffs[j], 32)
                u0 = urows[r, pl.ds(uoff, 16)]
                u1 = urows[r, pl.ds(uoff + 16, 16)]
                i0 = irows[r, pl.ds(ioff, 16)]
                i1 = irows[r, pl.ds(ioff + 16, 16)]
                p = u0 * i0 + u1 * i1
                for k in (8, 4, 2, 1):
                    p = p + _shuffle(p, (lanes + k) & 15)
                acc = jnp.where(lanes == j, p, acc)
            out_v[pl.ds(c * CHUNK + g * 16, 16)] = acc
            return carry

        lax.fori_loop(0, CHUNK // 16, stage, 0)

    pltpu.sync_copy(out_v, out_hbm.at[pl.ds(base, BPW)])


def kernel(user_ids, item_ids, user_table, item_table, user_bias, item_bias,
           global_bias):
    uid = user_ids.astype(jnp.int32)
    iid = item_ids.astype(jnp.int32)
    uq = (uid >> 2).reshape(NW * NCHUNK, CHUNK)
    iq = (iid >> 2).reshape(NW * NCHUNK, CHUNK)
    uo = (uid & 3) << 5
    io = (iid & 3) << 5
    utab = user_table.astype(jnp.float32).reshape(TROWS, 128)
    itab = item_table.astype(jnp.float32).reshape(TROWS, 128)
    mesh = plsc.VectorSubcoreMesh(core_axis_name="c", subcore_axis_name="s")
    f = pl.kernel(
        _body,
        mesh=mesh,
        compiler_params=pltpu.CompilerParams(use_tc_tiling_on_sc=False),
        out_type=jax.ShapeDtypeStruct((B,), jnp.float32),
        scratch_types=[
            pltpu.VMEM((NCHUNK, CHUNK), jnp.int32),
            pltpu.VMEM((NCHUNK, CHUNK), jnp.int32),
            pltpu.VMEM((BPW,), jnp.int32),
            pltpu.VMEM((BPW,), jnp.int32),
            pltpu.VMEM((CHUNK, 128), jnp.float32),
            pltpu.VMEM((CHUNK, 128), jnp.float32),
            pltpu.VMEM((BPW,), jnp.float32),
            pltpu.SemaphoreType.DMA,
            pltpu.SemaphoreType.DMA,
        ],
    )
    return f(uq, iq, uo, io, utab, itab)
